# Initial kernel scaffold; baseline (speedup 1.0000x reference)
import jax
import jax.numpy as jnp
from jax.experimental import pallas as pl


def kernel(k_val, v_val, k_cache, v_cache):
    raise NotImplementedError("write your pallas kernel here")


# speedup vs baseline: 3.5629x; 3.5629x over previous
import jax
import jax.numpy as jnp
from jax.experimental import pallas as pl


def _copy_body(k_ref, v_ref, ko_ref, vo_ref):
    ko_ref[...] = k_ref[...]
    vo_ref[...] = v_ref[...]


def kernel(k_val, v_val, k_cache, v_cache):
    B, H, S, D = k_val.shape
    k2 = k_val.reshape(B * H, S, D)
    v2 = v_val.reshape(B * H, S, D)
    spec = pl.BlockSpec((1, S, D), lambda i: (i, 0, 0))
    out = pl.pallas_call(
        _copy_body,
        grid=(B * H,),
        in_specs=[spec, spec],
        out_specs=[spec, spec],
        out_shape=[jax.ShapeDtypeStruct((B * H, S, D), k_val.dtype)] * 2,
    )(k2, v2)
    return out[0].reshape(B, H, S, D), out[1].reshape(B, H, S, D)
